# baseline (speedup 1.0000x reference)
import functools

import jax
import jax.numpy as jnp
from jax import lax
from jax.experimental import pallas as pl
from jax.experimental.pallas import tpu as pltpu
from jax.experimental.pallas import tpu_sc as plsc

NUM_CORES = 2
NUM_SUBCORES = 16
NW = NUM_CORES * NUM_SUBCORES
LANES = 16


@functools.lru_cache(maxsize=None)
def _build_gather(b: int, t: int, depth: int):
    assert b % 128 == 0 and depth % 8 == 0
    n_tiles = depth // 8
    n_bblk = b // 128
    chunk = 8 * 128 * n_bblk
    tiles_base = n_tiles // NW
    tiles_rem = n_tiles % NW
    max_tiles = tiles_base + (1 if tiles_rem else 0)
    assert t % 2 == 0

    mesh = plsc.VectorSubcoreMesh(
        core_axis_name="c", subcore_axis_name="s",
        num_cores=NUM_CORES, num_subcores=NUM_SUBCORES)

    @functools.partial(
        pl.kernel,
        mesh=mesh,
        compiler_params=pltpu.CompilerParams(
            use_tc_tiling_on_sc=False, needs_layout_passes=False),
        out_type=jax.ShapeDtypeStruct((t, n_tiles, chunk), jnp.float32),
        scratch_types=[
            pltpu.VMEM((max_tiles * 8, depth), jnp.float32),
            pltpu.VMEM((b,), jnp.int32),
            pltpu.VMEM((b,), jnp.int32),
            [pltpu.VMEM((chunk,), jnp.float32) for _ in range(max_tiles)],
            pltpu.SemaphoreType.DMA,
            pltpu.SemaphoreType.DMA,
            [pltpu.SemaphoreType.DMA for _ in range(max_tiles)],
        ],
    )
    def gather_kernel(idxT_hbm, tableT_hbm, out_hbm, slab, idx0, idx1,
                      obufs, isem0, isem1, osems):
        w = lax.axis_index("s") * NUM_CORES + lax.axis_index("c")
        lo = w * tiles_base + jnp.minimum(w, tiles_rem)
        nt = jnp.where(w < tiles_rem, tiles_base + 1, tiles_base)

        for k in range(max_tiles):
            @pl.when(k < nt)
            def _(k=k):
                pltpu.sync_copy(tableT_hbm.at[pl.ds((lo + k) * 8, 8)],
                                slab.at[pl.ds(k * 8, 8)])

        def compute_t(tt, cur):
            for k in range(max_tiles):
                buf, osem = obufs[k], osems[k]

                @pl.when(k < nt)
                def _(k=k, buf=buf, osem=osem):
                    @pl.when(tt > 0)
                    def _():
                        pltpu.make_async_copy(out_hbm.at[0, 0], buf, osem).wait()

                    @pl.loop(0, n_bblk, step=4)
                    def _bb(bb):
                        def load_ivs(blk):
                            return [cur[pl.ds(blk * 128 + j * LANES, LANES)]
                                    for j in range(128 // LANES)]

                        def store(blk, di, j, v):
                            buf[pl.ds(blk * 1024 + di * 128 + j * LANES,
                                      LANES)] = v

                        def body(blk, ivs, tail):
                            prev = [plsc.load_gather(slab.at[k * 8], [iv])
                                    for iv in ivs]
                            for di in range(1, 8):
                                row = slab.at[k * 8 + di]
                                curr = []
                                for j, iv in enumerate(ivs):
                                    store(blk, di - 1, j, prev[j])
                                    curr.append(plsc.load_gather(row, [iv]))
                                prev = curr
                            nxt = []
                            for j in range(len(ivs)):
                                store(blk, 7, j, prev[j])
                                if tail is not None:
                                    nxt.append(tail(j))
                            return nxt

                        def tail_for(blk):
                            return lambda j: cur[pl.ds(blk * 128 + j * LANES,
                                                       LANES)]

                        ivs = load_ivs(bb)
                        for u in range(3):
                            ivs = body(bb + u, ivs, tail_for(bb + u + 1))
                        body(bb + 3, ivs, None)

                    pltpu.async_copy(buf, out_hbm.at[tt, lo + k], osem)

        pltpu.async_copy(idxT_hbm.at[0], idx0, isem0)

        @pl.loop(0, t, step=2)
        def _tpair(tt):
            pltpu.make_async_copy(idxT_hbm.at[0], idx0, isem0).wait()

            @pl.when(tt + 1 < t)
            def _():
                pltpu.async_copy(idxT_hbm.at[tt + 1], idx1, isem1)

            compute_t(tt, idx0)
            pltpu.make_async_copy(idxT_hbm.at[0], idx1, isem1).wait()

            @pl.when(tt + 2 < t)
            def _():
                pltpu.async_copy(idxT_hbm.at[tt + 2], idx0, isem0)

            compute_t(tt + 1, idx1)

        for k in range(max_tiles):
            @pl.when(k < nt)
            def _(k=k):
                pltpu.make_async_copy(out_hbm.at[0, 0], obufs[k], osems[k]).wait()

    return gather_kernel


def kernel(idx, table):
    b, t = idx.shape
    _, depth = table.shape
    idx_t = idx.T.astype(jnp.int32)
    table_t = table.T
    x = _build_gather(b, t, depth)(idx_t, table_t)
    return (x.reshape(t, depth // 8, b // 128, 8, 128)
            .transpose(2, 4, 0, 1, 3).reshape(b, t, depth))


# speedup vs baseline: 1.2367x; 1.2367x over previous
import functools

import jax
import jax.numpy as jnp
from jax import lax
from jax.experimental import pallas as pl
from jax.experimental.pallas import tpu as pltpu
from jax.experimental.pallas import tpu_sc as plsc

NUM_CORES = 2
NUM_SUBCORES = 16
NW = NUM_CORES * NUM_SUBCORES
LANES = 16


@functools.lru_cache(maxsize=None)
def _build_gather(b: int, t: int, depth: int):
    assert b % 128 == 0 and depth % 8 == 0
    n_tiles = depth // 8
    n_bblk = b // 128
    chunk = 8 * 128 * n_bblk
    tiles_base = n_tiles // NW
    tiles_rem = n_tiles % NW
    max_tiles = tiles_base + (1 if tiles_rem else 0)
    assert t % 2 == 0

    mesh = plsc.VectorSubcoreMesh(
        core_axis_name="c", subcore_axis_name="s",
        num_cores=NUM_CORES, num_subcores=NUM_SUBCORES)

    @functools.partial(
        pl.kernel,
        mesh=mesh,
        compiler_params=pltpu.CompilerParams(
            use_tc_tiling_on_sc=False, needs_layout_passes=False),
        out_type=jax.ShapeDtypeStruct((t, n_tiles, chunk), jnp.float32),
        scratch_types=[
            pltpu.VMEM((max_tiles * 8, depth), jnp.float32),
            pltpu.VMEM((b,), jnp.int32),
            pltpu.VMEM((b,), jnp.int32),
            [pltpu.VMEM((chunk,), jnp.float32) for _ in range(max_tiles)],
            pltpu.SemaphoreType.DMA,
            pltpu.SemaphoreType.DMA,
            [pltpu.SemaphoreType.DMA for _ in range(max_tiles)],
        ],
    )
    def gather_kernel(idxT_hbm, tableT_hbm, out_hbm, slab, idx0, idx1,
                      obufs, isem0, isem1, osems):
        w = lax.axis_index("s") * NUM_CORES + lax.axis_index("c")
        lo = w * tiles_base + jnp.minimum(w, tiles_rem)
        nt = jnp.where(w < tiles_rem, tiles_base + 1, tiles_base)

        for k in range(max_tiles):
            @pl.when(k < nt)
            def _(k=k):
                pltpu.sync_copy(tableT_hbm.at[pl.ds((lo + k) * 8, 8)],
                                slab.at[pl.ds(k * 8, 8)])

        def compute_t(tt, cur):
            for k in range(max_tiles):
                buf, osem = obufs[k], osems[k]

                @pl.when(k < nt)
                def _(k=k, buf=buf, osem=osem):
                    @pl.when(tt > 0)
                    def _():
                        pltpu.make_async_copy(out_hbm.at[0, 0], buf, osem).wait()

                    @pl.loop(0, n_bblk, step=2)
                    def _bb(bb):
                        def load_ivs(blk):
                            return [cur[pl.ds(blk * 128 + j * LANES, LANES)]
                                    for j in range(128 // LANES)]

                        def store(blk, di, j, v):
                            buf[pl.ds(blk * 1024 + di * 128 + j * LANES,
                                      LANES)] = v

                        def body(blk, ivs, tail):
                            prev = [plsc.load_gather(slab.at[k * 8], [iv])
                                    for iv in ivs]
                            for di in range(1, 8):
                                row = slab.at[k * 8 + di]
                                curr = []
                                for j, iv in enumerate(ivs):
                                    store(blk, di - 1, j, prev[j])
                                    curr.append(plsc.load_gather(row, [iv]))
                                prev = curr
                            nxt = []
                            for j in range(len(ivs)):
                                store(blk, 7, j, prev[j])
                                if tail is not None:
                                    nxt.append(tail(j))
                            return nxt

                        ivs1 = body(bb, load_ivs(bb),
                                    lambda j: cur[pl.ds((bb + 1) * 128
                                                        + j * LANES, LANES)])
                        body(bb + 1, ivs1, None)

                    pltpu.async_copy(buf, out_hbm.at[tt, lo + k], osem)

        pltpu.async_copy(idxT_hbm.at[0], idx0, isem0)

        @pl.loop(0, t, step=2)
        def _tpair(tt):
            pltpu.make_async_copy(idxT_hbm.at[0], idx0, isem0).wait()

            @pl.when(tt + 1 < t)
            def _():
                pltpu.async_copy(idxT_hbm.at[tt + 1], idx1, isem1)

            compute_t(tt, idx0)
            pltpu.make_async_copy(idxT_hbm.at[0], idx1, isem1).wait()

            @pl.when(tt + 2 < t)
            def _():
                pltpu.async_copy(idxT_hbm.at[tt + 2], idx0, isem0)

            compute_t(tt + 1, idx1)

        for k in range(max_tiles):
            @pl.when(k < nt)
            def _(k=k):
                pltpu.make_async_copy(out_hbm.at[0, 0], obufs[k], osems[k]).wait()

    return gather_kernel


def kernel(idx, table):
    b, t = idx.shape
    _, depth = table.shape
    idx_t = idx.T.astype(jnp.int32)
    table_t = table.T
    x = _build_gather(b, t, depth)(idx_t, table_t)
    return (x.reshape(t, depth // 8, b // 128, 8, 128)
            .transpose(2, 4, 0, 1, 3).reshape(b, t, depth))


# speedup vs baseline: 1.2371x; 1.0003x over previous
import functools

import jax
import jax.numpy as jnp
from jax import lax
from jax.experimental import pallas as pl
from jax.experimental.pallas import tpu as pltpu
from jax.experimental.pallas import tpu_sc as plsc

NUM_CORES = 2
NUM_SUBCORES = 16
NW = NUM_CORES * NUM_SUBCORES
LANES = 16


@functools.lru_cache(maxsize=None)
def _build_gather(b: int, t: int, depth: int):
    assert b % 128 == 0 and depth % 8 == 0
    n_tiles = depth // 8
    n_bblk = b // 128
    chunk = 8 * 128 * n_bblk
    tiles_base = n_tiles // NW
    tiles_rem = n_tiles % NW
    max_tiles = tiles_base + (1 if tiles_rem else 0)
    assert t % 2 == 0 and n_bblk % 2 == 0

    mesh = plsc.VectorSubcoreMesh(
        core_axis_name="c", subcore_axis_name="s",
        num_cores=NUM_CORES, num_subcores=NUM_SUBCORES)

    @functools.partial(
        pl.kernel,
        mesh=mesh,
        compiler_params=pltpu.CompilerParams(
            use_tc_tiling_on_sc=False, needs_layout_passes=False),
        out_type=jax.ShapeDtypeStruct((t, n_tiles, chunk), jnp.float32),
        scratch_types=[
            pltpu.VMEM((max_tiles * 8, depth), jnp.float32),
            pltpu.VMEM((b,), jnp.int32),
            pltpu.VMEM((b,), jnp.int32),
            [pltpu.VMEM((chunk,), jnp.float32) for _ in range(max_tiles)],
            pltpu.SemaphoreType.DMA,
            pltpu.SemaphoreType.DMA,
            [pltpu.SemaphoreType.DMA for _ in range(max_tiles)],
        ],
    )
    def gather_kernel(idxT_hbm, tableT_hbm, out_hbm, slab, idx0, idx1,
                      obufs, isem0, isem1, osems):
        w = lax.axis_index("s") * NUM_CORES + lax.axis_index("c")
        lo = w * tiles_base + jnp.minimum(w, tiles_rem)
        nt = jnp.where(w < tiles_rem, tiles_base + 1, tiles_base)

        for k in range(max_tiles):
            @pl.when(k < nt)
            def _(k=k):
                pltpu.sync_copy(tableT_hbm.at[pl.ds((lo + k) * 8, 8)],
                                slab.at[pl.ds(k * 8, 8)])

        def compute_t(tt, cur):
            for k in range(max_tiles):
                buf, osem = obufs[k], osems[k]

                @pl.when(k < nt)
                def _(k=k, buf=buf, osem=osem):
                    @pl.when(tt > 0)
                    def _():
                        pltpu.make_async_copy(out_hbm.at[0, 0], buf, osem).wait()

                    @pl.loop(0, n_bblk, step=2)
                    def _bb(bb):
                        def load_ivs(blk):
                            return [cur[pl.ds(blk * 128 + j * LANES, LANES)]
                                    for j in range(128 // LANES)]

                        def store(blk, di, j, v):
                            buf[pl.ds(blk * 1024 + di * 128 + j * LANES,
                                      LANES)] = v

                        def body(blk, ivs, tail):
                            prev = [plsc.load_gather(slab.at[k * 8], [iv])
                                    for iv in ivs]
                            for di in range(1, 8):
                                row = slab.at[k * 8 + di]
                                curr = []
                                for j, iv in enumerate(ivs):
                                    store(blk, di - 1, j, prev[j])
                                    curr.append(plsc.load_gather(row, [iv]))
                                prev = curr
                            nxt = []
                            for j in range(len(ivs)):
                                store(blk, 7, j, prev[j])
                                if tail is not None:
                                    nxt.append(tail(j))
                            return nxt

                        ivs1 = body(bb, load_ivs(bb),
                                    lambda j: cur[pl.ds((bb + 1) * 128
                                                        + j * LANES, LANES)])
                        body(bb + 1, ivs1, None)

                    pltpu.async_copy(buf, out_hbm.at[tt, lo + k], osem)

        pltpu.async_copy(idxT_hbm.at[0], idx0, isem0)

        @pl.loop(0, t, step=2)
        def _tpair(tt):
            pltpu.make_async_copy(idxT_hbm.at[0], idx0, isem0).wait()

            @pl.when(tt + 1 < t)
            def _():
                pltpu.async_copy(idxT_hbm.at[tt + 1], idx1, isem1)

            compute_t(tt, idx0)
            pltpu.make_async_copy(idxT_hbm.at[0], idx1, isem1).wait()

            @pl.when(tt + 2 < t)
            def _():
                pltpu.async_copy(idxT_hbm.at[tt + 2], idx0, isem0)

            compute_t(tt + 1, idx1)

        for k in range(max_tiles):
            @pl.when(k < nt)
            def _(k=k):
                pltpu.make_async_copy(out_hbm.at[0, 0], obufs[k], osems[k]).wait()

    return gather_kernel


def kernel(idx, table):
    b, t = idx.shape
    _, depth = table.shape
    idx_t = idx.T.astype(jnp.int32)
    table_t = table.T
    x = _build_gather(b, t, depth)(idx_t, table_t)
    return (x.reshape(t, depth // 8, b // 128, 8, 128)
            .transpose(2, 4, 0, 1, 3).reshape(b, t, depth))
